# baseline (speedup 1.0000x reference)
import functools

import jax
import jax.numpy as jnp
from jax import lax
from jax.experimental import pallas as pl
from jax.experimental.pallas import tpu as pltpu
from jax.experimental.pallas import tpu_sc as plsc

_B = 16384
_D = 64
_NC = 2
_NS = 16
_NW = _NC * _NS
_BPW = _B // _NW
_CH = 128
_NCH = _BPW // _CH

_R = 2048
_TC_BLK = 2048


def _pack_body(in1_ref, in2_ref, in3_ref, in4_ref, eye_ref, out_ref):
    eye = eye_ref[...]
    tdot = lambda x: jax.lax.dot_general(
        x.astype(jnp.bfloat16), eye,
        dimension_numbers=(((0,), (0,)), ((), ())),
        preferred_element_type=jnp.float32)

    def pack2(lo, hi):
        ulo = jax.lax.bitcast_convert_type(lo, jnp.int32)
        uhi = jax.lax.bitcast_convert_type(hi, jnp.int32)
        word = jax.lax.shift_right_logical(ulo, 16) | (uhi & jnp.int32(-65536))
        return jax.lax.bitcast_convert_type(word, jnp.float32)

    out_ref[:, :_D] = pack2(tdot(in1_ref[...]), tdot(in2_ref[...]))
    out_ref[:, _D:] = pack2(tdot(in3_ref[...]), tdot(in4_ref[...]))


def _tc_transpose_pack(table_t, n_rows):
    n_groups = (n_rows + 4 * _R - 1) // (4 * _R)
    eye = jnp.eye(_D, dtype=jnp.bfloat16)
    return pl.pallas_call(
        _pack_body,
        grid=(n_groups,),
        in_specs=[
            pl.BlockSpec((_D, _R), lambda i: (0, 4 * i)),
            pl.BlockSpec((_D, _R), lambda i: (0, 4 * i + 1)),
            pl.BlockSpec((_D, _R), lambda i: (0, 4 * i + 2)),
            pl.BlockSpec((_D, _R), lambda i: (0, 4 * i + 3)),
            pl.BlockSpec((_D, _D), lambda i: (0, 0)),
        ],
        out_specs=pl.BlockSpec((_R, 2 * _D), lambda i: (i, 0)),
        out_shape=jax.ShapeDtypeStruct((n_groups * _R, 2 * _D), jnp.float32),
    )(table_t, table_t, table_t, table_t, eye)


def _sc_gather(obj_packed, attr_packed, oi_rows, ai_rows):
    mesh = plsc.VectorSubcoreMesh(core_axis_name="c", subcore_axis_name="s")

    @functools.partial(
        pl.kernel,
        mesh=mesh,
        out_type=(jax.ShapeDtypeStruct((_B, 2 * _D), jnp.float32),
                  jax.ShapeDtypeStruct((_B, 2 * _D), jnp.float32)),
        scratch_types=[
            pltpu.VMEM((_NCH, _CH), jnp.int32),
            pltpu.VMEM((_NCH, _CH), jnp.int32),
            pltpu.VMEM((_CH, 2 * _D), jnp.float32),
            pltpu.VMEM((_CH, 2 * _D), jnp.float32),
            pltpu.VMEM((_CH, 2 * _D), jnp.float32),
            pltpu.VMEM((_CH, 2 * _D), jnp.float32),
            pltpu.SemaphoreType.DMA,
            pltpu.SemaphoreType.DMA,
            pltpu.SemaphoreType.DMA,
            pltpu.SemaphoreType.DMA,
        ],
    )
    def gather_k(obj_hbm, attr_hbm, oi_hbm, ai_hbm, o_hbm, a_hbm,
                 oi_v, ai_v, orows0, arows0, orows1, arows1,
                 sem_o0, sem_o1, sem_a0, sem_a1):
        wid = lax.axis_index("s") * _NC + lax.axis_index("c")
        base = wid * _BPW
        pltpu.sync_copy(oi_hbm.at[pl.ds(wid * _NCH, _NCH)], oi_v)
        pltpu.sync_copy(ai_hbm.at[pl.ds(wid * _NCH, _NCH)], ai_v)

        obufs = (orows0, orows1)
        abufs = (arows0, arows1)
        osems = (sem_o0, sem_o1)
        asems = (sem_a0, sem_a1)
        cps = {}
        cps[0] = (pltpu.async_copy(obj_hbm.at[oi_v.at[0]], obufs[0], osems[0]),
                  pltpu.async_copy(attr_hbm.at[ai_v.at[0]], abufs[0], asems[0]))
        for c in range(_NCH):
            if c + 1 < _NCH:
                cps[c + 1] = (
                    pltpu.async_copy(obj_hbm.at[oi_v.at[c + 1]],
                                     obufs[(c + 1) % 2], osems[(c + 1) % 2]),
                    pltpu.async_copy(attr_hbm.at[ai_v.at[c + 1]],
                                     abufs[(c + 1) % 2], asems[(c + 1) % 2]))
            cp_o, cp_a = cps.pop(c)
            cp_o.wait()
            pltpu.sync_copy(obufs[c % 2], o_hbm.at[pl.ds(base + c * _CH, _CH)])
            cp_a.wait()
            pltpu.sync_copy(abufs[c % 2], a_hbm.at[pl.ds(base + c * _CH, _CH)])

    return gather_k(obj_packed, attr_packed,
                    oi_rows.reshape(_NW * _NCH, _CH),
                    ai_rows.reshape(_NW * _NCH, _CH))


def _dense_body(go_ref, ga_ref, s_ref, rw_ref, rb_ref,
                fw_ref, fb_ref, c1o_ref, c1a_ref, c1b_ref,
                c2w_ref, c2b_ref,
                shared_ref, route_ref, context_ref, out_ref):
    def unpack(g_ref, s):
        half = jnp.where(s >= 2, g_ref[:, _D:], g_ref[:, :_D])
        u = jax.lax.bitcast_convert_type(half, jnp.int32)
        bits = jnp.where(s % 2 == 1, u & jnp.int32(-65536),
                         jax.lax.shift_left(u, 16))
        return jax.lax.bitcast_convert_type(bits, jnp.float32)

    s = s_ref[...]
    o = unpack(go_ref, s % 4).astype(jnp.bfloat16)
    a = unpack(ga_ref, s // 4).astype(jnp.bfloat16)
    bdot = lambda x, w: jnp.dot(x, w.astype(jnp.bfloat16),
                                preferred_element_type=jnp.float32)
    route = jax.nn.sigmoid(bdot(o, rw_ref[...]) + rb_ref[...])
    shared = jax.nn.sigmoid(bdot(a, fw_ref[...]) + fb_ref[...])
    h = jnp.tanh(bdot(o, c1o_ref[...]) + bdot(a, c1a_ref[...]) + c1b_ref[...])
    context = jax.nn.sigmoid(
        bdot(h.astype(jnp.bfloat16), c2w_ref[...]) + c2b_ref[...])
    sharedt = shared.T
    routet = route.T
    contextt = context.T
    shared_ref[...] = sharedt
    route_ref[...] = routet
    context_ref[...] = contextt
    out_ref[...] = jnp.concatenate([sharedt, routet[1:2, :], contextt], axis=0)


def _tc_dense(g_o, g_a, sel_w, router_W, router_b, fiber_W, fiber_b,
              c1_W, c1_b, c2_W, c2_b):
    grid = (_B // _TC_BLK,)
    row_spec = lambda w: pl.BlockSpec((_TC_BLK, w), lambda i: (i, 0))
    col_spec = lambda h: pl.BlockSpec((h, _TC_BLK), lambda i: (0, i))
    full_spec = lambda s: pl.BlockSpec(s, lambda i: (0, 0))
    return pl.pallas_call(
        _dense_body,
        grid=grid,
        in_specs=[
            row_spec(2 * _D), row_spec(2 * _D), row_spec(1),
            full_spec((_D, 2)), full_spec((1, 2)),
            full_spec((_D, 1)), full_spec((1, 1)),
            full_spec((_D, _D)), full_spec((_D, _D)), full_spec((1, _D)),
            full_spec((_D, 1)), full_spec((1, 1)),
        ],
        out_specs=[col_spec(1), col_spec(2), col_spec(1), col_spec(3)],
        out_shape=[
            jax.ShapeDtypeStruct((1, _B), jnp.float32),
            jax.ShapeDtypeStruct((2, _B), jnp.float32),
            jax.ShapeDtypeStruct((1, _B), jnp.float32),
            jax.ShapeDtypeStruct((3, _B), jnp.float32),
        ],
    )(g_o, g_a, sel_w,
      router_W, router_b.reshape(1, 2),
      fiber_W, fiber_b.reshape(1, 1),
      c1_W[:_D], c1_W[_D:], c1_b.reshape(1, _D),
      c2_W, c2_b.reshape(1, 1))


def kernel(object_idx, attr_idx, object_table, attr_table,
           router_W, router_b, fiber_W, fiber_b,
           c1_W, c1_b, c2_W, c2_b):
    oi = object_idx.astype(jnp.int32)
    ai = attr_idx.astype(jnp.int32)
    oi_rows = (oi // (4 * _R)) * _R + (oi % _R)
    ai_rows = (ai // (4 * _R)) * _R + (ai % _R)
    sel_w = (((oi // _R) % 4) + 4 * ((ai // _R) % 4)).astype(
        jnp.int32).reshape(_B, 1)

    obj_packed = _tc_transpose_pack(object_table.T, object_table.shape[0])
    attr_packed = _tc_transpose_pack(attr_table.T, attr_table.shape[0])
    g_o, g_a = _sc_gather(obj_packed, attr_packed, oi_rows, ai_rows)
    sharedt, routet, contextt, outt = _tc_dense(
        g_o, g_a, sel_w, router_W, router_b, fiber_W, fiber_b,
        c1_W, c1_b, c2_W, c2_b)
    return (sharedt.T, routet.T, contextt.T, outt.T)
